# Initial kernel scaffold; baseline (speedup 1.0000x reference)
import jax
import jax.numpy as jnp
from jax.experimental import pallas as pl


def kernel(x, weights, indices, fc1_weights, fc2_weights):
    raise NotImplementedError("write your pallas kernel here")


# speedup vs baseline: 3.2125x; 3.2125x over previous
import functools

import jax
import jax.numpy as jnp
from jax import lax
from jax.experimental import pallas as pl
from jax.experimental.pallas import tpu as pltpu
from jax.experimental.pallas import tpu_sc as plsc

_B = 128
_CHUNK = 16


def _sc_row_gather(src, idx, n_out):
    info = plsc.get_sparse_core_info()
    nw = info.num_cores * info.num_subcores
    d = src.shape[1]
    assert n_out % (nw * _CHUNK) == 0
    per_w = n_out // nw
    n_chunks = per_w // _CHUNK

    mesh = plsc.VectorSubcoreMesh(core_axis_name="c", subcore_axis_name="s")

    @functools.partial(
        pl.kernel,
        mesh=mesh,
        out_type=jax.ShapeDtypeStruct((n_out, d), src.dtype),
        scratch_types=[
            pltpu.VMEM((_CHUNK,), jnp.int32),
            pltpu.VMEM((_CHUNK, d), src.dtype),
            pltpu.SemaphoreType.DMA,
        ],
    )
    def k(src_hbm, idx_hbm, out_hbm, idx_v, rows_v, sem):
        wid = lax.axis_index("s") * info.num_cores + lax.axis_index("c")
        base0 = wid * per_w

        def body(c, _):
            base = base0 + c * _CHUNK
            pltpu.sync_copy(idx_hbm.at[pl.ds(base, _CHUNK)], idx_v)
            pltpu.async_copy(src_hbm.at[idx_v], rows_v, sem).wait()
            pltpu.sync_copy(rows_v, out_hbm.at[pl.ds(base, _CHUNK)])
            return ()

        lax.fori_loop(0, n_chunks, body, (), unroll=False)

    return k(src, idx)


def _mm_kernel(b2e_ref, nused_ref, x_ref, w1_ref, w2_ref, ws_ref, o_ref):
    i = pl.program_id(0)

    @pl.when(i < nused_ref[0])
    def _():
        xb = x_ref[...]
        w1 = w1_ref[0]
        h = lax.dot_general(xb, w1, (((1,), (1,)), ((), ())),
                            preferred_element_type=jnp.float32)
        f = w1.shape[0] // 2
        y = h[:, :f]
        g = h[:, f:]
        act = y * (g * jax.nn.sigmoid(g))
        w2 = w2_ref[0]
        ob = lax.dot_general(act, w2, (((1,), (1,)), ((), ())),
                             preferred_element_type=jnp.float32)
        o_ref[...] = ob * ws_ref[...]


def _grouped_mlp(x_sorted, fc1, fc2, w_sorted, b2e, num_used, num_blocks):
    p_max, d_model = x_sorted.shape
    e, two_ff, _ = fc1.shape
    d_ff = two_ff // 2

    def last_used(i, nu):
        return jnp.minimum(i, nu[0] - 1)

    grid_spec = pltpu.PrefetchScalarGridSpec(
        num_scalar_prefetch=2,
        grid=(num_blocks,),
        in_specs=[
            pl.BlockSpec((_B, d_model), lambda i, be, nu: (last_used(i, nu), 0)),
            pl.BlockSpec((1, two_ff, d_model),
                         lambda i, be, nu: (be[last_used(i, nu)], 0, 0)),
            pl.BlockSpec((1, d_model, d_ff),
                         lambda i, be, nu: (be[last_used(i, nu)], 0, 0)),
            pl.BlockSpec((_B, 1), lambda i, be, nu: (last_used(i, nu), 0)),
        ],
        out_specs=pl.BlockSpec((_B, d_model),
                               lambda i, be, nu: (last_used(i, nu), 0)),
    )
    return pl.pallas_call(
        _mm_kernel,
        grid_spec=grid_spec,
        out_shape=jax.ShapeDtypeStruct((p_max, d_model), jnp.float32),
        compiler_params=pltpu.CompilerParams(
            dimension_semantics=("arbitrary",),
            vmem_limit_bytes=100 * 1024 * 1024,
        ),
    )(b2e, num_used, x_sorted, fc1, fc2, w_sorted)


def kernel(x, weights, indices, fc1_weights, fc2_weights):
    t, d_model = x.shape
    e = fc1_weights.shape[0]
    p_max = t + e * _B
    num_blocks = t // _B + e

    flat = indices.reshape(-1).astype(jnp.int32)
    sort_ids = jnp.argsort(flat, stable=True).astype(jnp.int32)
    sorted_flat = jnp.take(flat, sort_ids)
    bounds = jnp.searchsorted(
        sorted_flat, jnp.arange(e + 1, dtype=jnp.int32), side="left"
    ).astype(jnp.int32)
    counts = bounds[1:] - bounds[:-1]
    nblk = (counts + _B - 1) // _B
    starts = jnp.concatenate(
        [jnp.zeros((1,), jnp.int32), jnp.cumsum(nblk * _B).astype(jnp.int32)]
    )
    p_used = starts[e]
    num_used = (p_used // _B).reshape(1).astype(jnp.int32)

    p = jnp.arange(p_max, dtype=jnp.int32)
    e_of_p = jnp.minimum(
        jnp.searchsorted(starts[1:], p, side="right").astype(jnp.int32), e - 1)
    local = p - jnp.take(starts, e_of_p)
    r = jnp.take(bounds, e_of_p) + local
    valid = local < jnp.take(counts, e_of_p)
    src = jnp.where(valid, jnp.take(sort_ids, jnp.clip(r, 0, t - 1)), 0)
    src = src.astype(jnp.int32)

    inv_rank = jnp.argsort(sort_ids).astype(jnp.int32)
    pos = (jnp.take(starts, flat) + (inv_rank - jnp.take(bounds, flat))
           ).astype(jnp.int32)

    blk_p = jnp.arange(num_blocks, dtype=jnp.int32) * _B
    b2e = jnp.minimum(
        jnp.searchsorted(starts[1:], blk_p, side="right").astype(jnp.int32),
        e - 1)

    w_sorted = jnp.take(weights[:, 0], src).reshape(p_max, 1)

    x_sorted = _sc_row_gather(x, src, p_max)
    y_sorted = _grouped_mlp(x_sorted, fc1_weights, fc2_weights, w_sorted,
                            b2e, num_used, num_blocks)
    out = _sc_row_gather(y_sorted, pos, t)
    return out
